# baseline (speedup 1.0000x reference)
import jax
import jax.numpy as jnp
from jax.experimental import pallas as pl
from jax.experimental.pallas import tpu as pltpu

_MAX_WEIGHT_NORM = 0.01
_NCH = 16


def _make_body(rows, cols):
    rch = rows // _NCH

    def body(sp_hbm, cm_hbm, out_hbm, sp_v, cm_v, sp_sems, cm_sems, out_sems):
        def chunk(ref, i):
            return ref.at[pl.ds(i * rch, rch)]

        sp_dmas = [
            pltpu.make_async_copy(chunk(sp_hbm, i), chunk(sp_v, i), sp_sems.at[i])
            for i in range(_NCH)
        ]
        cm_dmas = [
            pltpu.make_async_copy(chunk(cm_hbm, i), chunk(cm_v, i), cm_sems.at[i])
            for i in range(_NCH)
        ]
        for d in sp_dmas:
            d.start()
        for d in cm_dmas:
            d.start()

        acc = jnp.zeros((rch, cols), jnp.float32)
        for i in range(_NCH):
            sp_dmas[i].wait()
            s = sp_v[pl.ds(i * rch, rch), :]
            acc = acc + s * s
        ssq = jnp.sum(acc)

        norm = jnp.sqrt(ssq)
        scale = jnp.where(norm > _MAX_WEIGHT_NORM, _MAX_WEIGHT_NORM / norm, 1.0)

        out_dmas = [
            pltpu.make_async_copy(chunk(sp_v, i), chunk(out_hbm, i), out_sems.at[i])
            for i in range(_NCH)
        ]
        for i in range(_NCH):
            cm_dmas[i].wait()
            sl = (pl.ds(i * rch, rch), slice(None))
            sp_v[sl] = sp_v[sl] * scale + cm_v[sl]
            out_dmas[i].start()
        for d in out_dmas:
            d.wait()

    return body


def kernel(x, lens, common_latent, special_latent):
    del x, lens
    rows, cols = special_latent.shape
    return pl.pallas_call(
        _make_body(rows, cols),
        in_specs=[
            pl.BlockSpec(memory_space=pltpu.HBM),
            pl.BlockSpec(memory_space=pltpu.HBM),
        ],
        out_specs=pl.BlockSpec(memory_space=pltpu.HBM),
        out_shape=jax.ShapeDtypeStruct((rows, cols), special_latent.dtype),
        scratch_shapes=[
            pltpu.VMEM((rows, cols), jnp.float32),
            pltpu.VMEM((rows, cols), jnp.float32),
            pltpu.SemaphoreType.DMA((_NCH,)),
            pltpu.SemaphoreType.DMA((_NCH,)),
            pltpu.SemaphoreType.DMA((_NCH,)),
        ],
        compiler_params=pltpu.CompilerParams(
            disable_bounds_checks=True,
            disable_semaphore_checks=True,
            skip_device_barrier=True,
        ),
    )(special_latent, common_latent)


# speedup vs baseline: 1.0076x; 1.0076x over previous
import jax
import jax.numpy as jnp
from jax.experimental import pallas as pl
from jax.experimental.pallas import tpu as pltpu

_MAX_WEIGHT_NORM = 0.01
_NCH = 8


def _make_body(rows, cols):
    rch = rows // _NCH

    def body(sp_hbm, cm_hbm, out_hbm, sp_v, cm_v, sp_sems, cm_sems, out_sems):
        def chunk(ref, i):
            return ref.at[pl.ds(i * rch, rch)]

        sp_dmas = [
            pltpu.make_async_copy(chunk(sp_hbm, i), chunk(sp_v, i), sp_sems.at[i])
            for i in range(_NCH)
        ]
        cm_dmas = [
            pltpu.make_async_copy(chunk(cm_hbm, i), chunk(cm_v, i), cm_sems.at[i])
            for i in range(_NCH)
        ]
        for d in sp_dmas:
            d.start()
        for d in cm_dmas:
            d.start()

        acc = jnp.zeros((rch, cols), jnp.float32)
        for i in range(_NCH):
            sp_dmas[i].wait()
            s = sp_v[pl.ds(i * rch, rch), :]
            acc = acc + s * s
        ssq = jnp.sum(acc)

        norm = jnp.sqrt(ssq)
        scale = jnp.where(norm > _MAX_WEIGHT_NORM, _MAX_WEIGHT_NORM / norm, 1.0)

        out_dmas = [
            pltpu.make_async_copy(chunk(sp_v, i), chunk(out_hbm, i), out_sems.at[i])
            for i in range(_NCH)
        ]
        for i in range(_NCH):
            cm_dmas[i].wait()
            sl = (pl.ds(i * rch, rch), slice(None))
            sp_v[sl] = sp_v[sl] * scale + cm_v[sl]
            out_dmas[i].start()
        for d in out_dmas:
            d.wait()

    return body


def kernel(x, lens, common_latent, special_latent):
    del x, lens
    rows, cols = special_latent.shape
    return pl.pallas_call(
        _make_body(rows, cols),
        in_specs=[
            pl.BlockSpec(memory_space=pltpu.HBM),
            pl.BlockSpec(memory_space=pltpu.HBM),
        ],
        out_specs=pl.BlockSpec(memory_space=pltpu.HBM),
        out_shape=jax.ShapeDtypeStruct((rows, cols), special_latent.dtype),
        scratch_shapes=[
            pltpu.VMEM((rows, cols), jnp.float32),
            pltpu.VMEM((rows, cols), jnp.float32),
            pltpu.SemaphoreType.DMA((_NCH,)),
            pltpu.SemaphoreType.DMA((_NCH,)),
            pltpu.SemaphoreType.DMA((_NCH,)),
        ],
        compiler_params=pltpu.CompilerParams(
            disable_bounds_checks=True,
            disable_semaphore_checks=True,
            skip_device_barrier=True,
        ),
    )(special_latent, common_latent)


# speedup vs baseline: 1.0142x; 1.0066x over previous
import jax
import jax.numpy as jnp
from jax.experimental import pallas as pl
from jax.experimental.pallas import tpu as pltpu

_MAX_WEIGHT_NORM = 0.01
_NCH = 8


def _make_body(rows, cols):
    rch = rows // _NCH

    def body(sp_hbm, cm_hbm, out_hbm, sp_v, cm_v, o_v, sp_sems, cm_sems, out_sems):
        def chunk(ref, i):
            return ref.at[pl.ds(i * rch, rch)]

        sp_dmas = [
            pltpu.make_async_copy(chunk(sp_hbm, i), chunk(sp_v, i), sp_sems.at[i])
            for i in range(_NCH)
        ]
        cm_dmas = [
            pltpu.make_async_copy(chunk(cm_hbm, i), chunk(cm_v, i), cm_sems.at[i])
            for i in range(_NCH)
        ]
        for d in sp_dmas:
            d.start()
        for d in cm_dmas:
            d.start()

        acc = jnp.zeros((rch, cols), jnp.float32)
        for i in range(_NCH):
            sp_dmas[i].wait()
            s = sp_v[pl.ds(i * rch, rch), :]
            acc = acc + s * s
        ssq = jnp.sum(acc)

        norm = jnp.sqrt(ssq)
        scale = jnp.where(norm > _MAX_WEIGHT_NORM, _MAX_WEIGHT_NORM / norm, 1.0)

        out_dmas = [
            pltpu.make_async_copy(chunk(o_v, i), chunk(out_hbm, i), out_sems.at[i])
            for i in range(_NCH)
        ]
        for i in range(_NCH):
            cm_dmas[i].wait()
            sl = (pl.ds(i * rch, rch), slice(None))
            o_v[sl] = sp_v[sl] * scale + cm_v[sl]
            out_dmas[i].start()
        for d in out_dmas:
            d.wait()

    return body


def kernel(x, lens, common_latent, special_latent):
    del x, lens
    rows, cols = special_latent.shape
    return pl.pallas_call(
        _make_body(rows, cols),
        in_specs=[
            pl.BlockSpec(memory_space=pltpu.HBM),
            pl.BlockSpec(memory_space=pltpu.HBM),
        ],
        out_specs=pl.BlockSpec(memory_space=pltpu.HBM),
        out_shape=jax.ShapeDtypeStruct((rows, cols), special_latent.dtype),
        scratch_shapes=[
            pltpu.VMEM((rows, cols), jnp.float32),
            pltpu.VMEM((rows, cols), jnp.float32),
            pltpu.VMEM((rows, cols), jnp.float32),
            pltpu.SemaphoreType.DMA((_NCH,)),
            pltpu.SemaphoreType.DMA((_NCH,)),
            pltpu.SemaphoreType.DMA((_NCH,)),
        ],
        compiler_params=pltpu.CompilerParams(
            disable_bounds_checks=True,
            disable_semaphore_checks=True,
            skip_device_barrier=True,
        ),
    )(special_latent, common_latent)
